# baseline (speedup 1.0000x reference)
import jax
import jax.numpy as jnp
from jax.experimental import pallas as pl

_B = 4
_N = 4096
_D = 4096
_K = 512
_ROWS_PER_BLK = 512


def _matvec_body(x_ref, rt_ref, out_ref):
    xb = x_ref[...].astype(jnp.bfloat16).astype(jnp.float32)
    rt = rt_ref[...].astype(jnp.bfloat16).astype(jnp.float32)
    out_ref[...] = jnp.sum(xb * rt, axis=1, keepdims=True)


def _select_body(srow_ref, scol_ref, idx_ref, ones_ref):
    s_row = srow_ref[0]
    start = _N - _K

    def chunk(ci, acc):
        s_chunk = scol_ref[0, pl.ds(ci * 128, 128), :]
        lt = (s_row < s_chunk).astype(jnp.int32)
        rc = jnp.sum(lt, axis=1, keepdims=True)
        p = jax.lax.broadcasted_iota(jnp.int32, (128, _K), 1)
        i_mat = jax.lax.broadcasted_iota(jnp.int32, (128, _K), 0) + ci * 128
        hit = (rc - start) == p
        return acc + jnp.sum(jnp.where(hit, i_mat, 0), axis=0, keepdims=True)

    acc = jax.lax.fori_loop(0, _N // 128, chunk, jnp.zeros((1, _K), jnp.int32))
    idx_ref[0] = acc
    ones_ref[0] = jnp.ones((1, _K), jnp.float32)


def kernel(x, routing_token, num_tokens):
    del num_tokens
    xf = x.reshape(_B * _N, _D)
    rt = routing_token.reshape(1, _D)

    n_blocks = (_B * _N) // _ROWS_PER_BLK
    scores_col = pl.pallas_call(
        _matvec_body,
        grid=(n_blocks,),
        in_specs=[
            pl.BlockSpec((_ROWS_PER_BLK, _D), lambda g: (g, 0)),
            pl.BlockSpec((1, _D), lambda g: (0, 0)),
        ],
        out_specs=pl.BlockSpec((_ROWS_PER_BLK, 1), lambda g: (g, 0)),
        out_shape=jax.ShapeDtypeStruct((_B * _N, 1), jnp.float32),
    )(xf, rt)

    scores = scores_col.reshape(_B, _N)
    s_row = scores.reshape(_B, 1, _N)
    s_col = scores.reshape(_B, _N, 1)

    sel_idx, sel_ones = pl.pallas_call(
        _select_body,
        grid=(_B,),
        in_specs=[
            pl.BlockSpec((1, 1, _N), lambda b: (b, 0, 0)),
            pl.BlockSpec((1, _N, 1), lambda b: (b, 0, 0)),
        ],
        out_specs=[
            pl.BlockSpec((1, 1, _K), lambda b: (b, 0, 0)),
            pl.BlockSpec((1, 1, _K), lambda b: (b, 0, 0)),
        ],
        out_shape=[
            jax.ShapeDtypeStruct((_B, 1, _K), jnp.int32),
            jax.ShapeDtypeStruct((_B, 1, _K), jnp.float32),
        ],
    )(s_row, s_col)

    return (sel_ones.reshape(_B, _K), sel_idx.reshape(_B, _K))
